# baseline (speedup 1.0000x reference)
import functools

import jax
import jax.numpy as jnp
from jax import lax
from jax.experimental import pallas as pl
from jax.experimental.pallas import tpu as pltpu
from jax.experimental.pallas import tpu_sc as plsc

N = 10000
E = 320000
D = 128

NC = 2
NS = 16
NW = NC * NS
CW = 64
NCHUNK = 162
EPT = NCHUNK * CW
EP = EPT * NW
NA = N + 16
RZ = 624

_PREC = jax.lax.Precision.DEFAULT

_MESH = plsc.VectorSubcoreMesh(core_axis_name="c", subcore_axis_name="s")


def _zero_acc(zeros, acc, s):
  pltpu.sync_copy(zeros.at[pl.ds(s * RZ, RZ)], acc.at[pl.ds(s * RZ, RZ)])

  @pl.when(s == 0)
  def _():
    pltpu.sync_copy(zeros.at[pl.ds(0, NA - NS * RZ)],
                    acc.at[pl.ds(NS * RZ, NA - NS * RZ)])


def _writeout(acc, out, c, s):
  pltpu.sync_copy(acc.at[pl.ds(s * RZ, RZ)], out.at[c, pl.ds(s * RZ, RZ)])

  @pl.when(s == 0)
  def _():
    pltpu.sync_copy(acc.at[pl.ds(NS * RZ, N - NS * RZ)],
                    out.at[c, pl.ds(NS * RZ, N - NS * RZ)])


def _conv_phase(table, src2d, dst2d, row0, sidx, didx, rows, agg_sh,
                isems, gsems, ssems):
  def istart(j, b):
    pltpu.async_copy(src2d.at[row0 + j], sidx[b], isems[b])
    pltpu.async_copy(dst2d.at[row0 + j], didx[b], isems[b])

  def iwait(b):
    pltpu.make_async_copy(src2d.at[0], sidx[b], isems[b]).wait()
    pltpu.make_async_copy(dst2d.at[0], didx[b], isems[b]).wait()

  def gstart(b3, b6):
    pltpu.async_copy(table.at[sidx[b6]], rows.at[b3], gsems[b3])

  def gwait(b3):
    pltpu.make_async_copy(table.at[pl.ds(0, CW)], rows.at[b3],
                          gsems[b3]).wait()

  def sstart(b3, b6):
    pltpu.async_copy(rows.at[b3], agg_sh.at[didx[b6]], ssems[b3], add=True)

  def swait(b3):
    pltpu.make_async_copy(rows.at[b3], agg_sh.at[pl.ds(0, CW)],
                          ssems[b3]).wait()

  for j in range(4):
    istart(j, j)
  iwait(0)
  gstart(0, 0)
  iwait(1)
  gstart(1, 1)

  plsc.subcore_barrier()

  def body(i, carry):
    j0 = 6 * i
    for b in range(6):
      j = j0 + b

      @pl.when(j + 2 < NCHUNK)
      def _():
        iwait((b + 2) % 6)
        @pl.when(j >= 1)
        def _():
          swait((b + 2) % 3)

        gstart((b + 2) % 3, (b + 2) % 6)

      gwait(b % 3)
      sstart(b % 3, b)

      @pl.when(j + 4 < NCHUNK)
      def _():
        istart(j + 4, (b + 4) % 6)
    return carry

  lax.fori_loop(0, NCHUNK // 6, body, 0)
  for b3 in range(3):
    swait(b3)


@functools.partial(
    pl.kernel,
    out_type=[
        jax.ShapeDtypeStruct((NC, N, D), jnp.float32),
        jax.ShapeDtypeStruct((NC, N, D), jnp.float32),
    ],
    mesh=_MESH,
    scratch_types=[
        [pltpu.VMEM((CW,), jnp.int32) for _ in range(6)],
        [pltpu.VMEM((CW,), jnp.int32) for _ in range(6)],
        pltpu.VMEM((3, CW, D), jnp.float32),
        pltpu.VMEM((CW, D), jnp.float32),
        pltpu.VMEM_SHARED((NA, D), jnp.float32),
        [pltpu.SemaphoreType.DMA for _ in range(6)],
        [pltpu.SemaphoreType.DMA for _ in range(3)],
        [pltpu.SemaphoreType.DMA for _ in range(3)],
    ],
)
def _conv1_deg(table, src2d, dst2d, zeros, ones, out, out_deg, sidx, didx,
               rows, ones_v, agg_sh, isems, gsems, ssems):
  c = lax.axis_index("c")
  s = lax.axis_index("s")
  row0 = (c * NS + s) * NCHUNK
  _zero_acc(zeros, agg_sh, s)
  pltpu.sync_copy(ones, ones_v)

  def istart_d(j, b):
    pltpu.async_copy(dst2d.at[row0 + j], didx[b], isems[b])

  def iwait_d(b):
    pltpu.make_async_copy(dst2d.at[0], didx[b], isems[b]).wait()

  def swait_d(b3):
    pltpu.make_async_copy(ones_v, agg_sh.at[pl.ds(0, CW)], ssems[b3]).wait()

  for j in range(4):
    istart_d(j, j)
  plsc.subcore_barrier()

  def dbody(i, carry):
    j0 = 6 * i
    for b in range(6):
      j = j0 + b

      @pl.when(j >= 2)
      def _():
        swait_d((b + 1) % 3)

      iwait_d(b)
      pltpu.async_copy(ones_v, agg_sh.at[didx[b]], ssems[b % 3], add=True)

      @pl.when(j + 4 < NCHUNK)
      def _():
        istart_d(j + 4, (b + 4) % 6)
    return carry

  lax.fori_loop(0, NCHUNK // 6, dbody, 0)
  for b3 in ((NCHUNK - 2) % 3, (NCHUNK - 1) % 3):
    swait_d(b3)
  plsc.subcore_barrier()

  _writeout(agg_sh, out_deg, c, s)
  _zero_acc(zeros, agg_sh, s)

  _conv_phase(table, src2d, dst2d, row0, sidx, didx, rows, agg_sh,
              isems, gsems, ssems)

  plsc.subcore_barrier()
  _writeout(agg_sh, out, c, s)


@functools.partial(
    pl.kernel,
    out_type=jax.ShapeDtypeStruct((NC, N, D), jnp.float32),
    mesh=_MESH,
    scratch_types=[
        [pltpu.VMEM((CW,), jnp.int32) for _ in range(6)],
        [pltpu.VMEM((CW,), jnp.int32) for _ in range(6)],
        pltpu.VMEM((3, CW, D), jnp.float32),
        pltpu.VMEM_SHARED((NA, D), jnp.float32),
        [pltpu.SemaphoreType.DMA for _ in range(6)],
        [pltpu.SemaphoreType.DMA for _ in range(3)],
        [pltpu.SemaphoreType.DMA for _ in range(3)],
    ],
)
def _conv(table, src2d, dst2d, zeros, out, sidx, didx, rows, agg_sh,
          isems, gsems, ssems):
  c = lax.axis_index("c")
  s = lax.axis_index("s")
  row0 = (c * NS + s) * NCHUNK
  _zero_acc(zeros, agg_sh, s)
  _conv_phase(table, src2d, dst2d, row0, sidx, didx, rows, agg_sh,
              isems, gsems, ssems)
  plsc.subcore_barrier()
  _writeout(agg_sh, out, c, s)


BLK = 1000
GRID = N // BLK


def _tc1_body(h_ref, p_ref, deg_ref, Ws_ref, Wn_ref, b_ref, g_ref, be_ref,
              h1_ref, inv_ref):
  h = h_ref[...]
  p = p_ref[...]
  agg = p[0] + p[1]
  deg = deg_ref[0, :, :1] + deg_ref[1, :, :1]
  inv = 1.0 / jnp.maximum(deg, 1.0)
  z = (jnp.dot(h, Ws_ref[...], precision=_PREC)
       + jnp.dot(agg * inv, Wn_ref[...], precision=_PREC) + b_ref[...])
  mu = jnp.mean(z, axis=-1, keepdims=True)
  zc = z - mu
  var = jnp.mean(zc * zc, axis=-1, keepdims=True)
  zn = zc / jnp.sqrt(var + 1e-5) * g_ref[...] + be_ref[...]
  h1_ref[...] = jnp.maximum(zn, 0.0)
  inv_ref[...] = inv


def _tc2_body(h_ref, p_ref, inv_ref, Ws_ref, Wn_ref, b_ref, g_ref, be_ref,
              Wl0_ref, bl0_ref, Wl1_ref, bl1_ref, Wh_ref, bh_ref, out_ref):
  h = h_ref[...]
  p = p_ref[...]
  agg = (p[0] + p[1]) * inv_ref[...]
  z = (jnp.dot(h, Ws_ref[...], precision=_PREC)
       + jnp.dot(agg, Wn_ref[...], precision=_PREC) + b_ref[...])
  mu = jnp.mean(z, axis=-1, keepdims=True)
  zc = z - mu
  var = jnp.mean(zc * zc, axis=-1, keepdims=True)
  zn = zc / jnp.sqrt(var + 1e-5) * g_ref[...] + be_ref[...]
  h2 = jnp.maximum(zn, 0.0)
  z0 = jnp.maximum(jnp.dot(h2, Wl0_ref[...], precision=_PREC)
                   + bl0_ref[...], 0.0)
  z1 = jnp.maximum(jnp.dot(z0, Wl1_ref[...], precision=_PREC)
                   + bl1_ref[...], 0.0)
  out_ref[...] = jnp.dot(z1, Wh_ref[...], precision=_PREC) + bh_ref[...]


def _full(shape):
  nd = len(shape)
  return pl.BlockSpec(shape, lambda i: (0,) * nd)


def kernel(x, edge_index, W_self0, W_nei0, b0, g0, be0, W_self1, W_nei1, b1,
           g1, be1, W_lin0, bl0, W_lin1, bl1, W_head, b_head):
  npad = EP - E
  pad_iota = jnp.arange(npad, dtype=jnp.int32)
  src_p = jnp.concatenate([edge_index[0], pad_iota % N])
  dst_p = jnp.concatenate([edge_index[1], N + (pad_iota % (NA - N))])
  src2d = src_p.reshape(EP // CW, CW)
  dst2d = dst_p.reshape(EP // CW, CW)
  z128 = jnp.zeros((N, D), jnp.float32)
  ones = jnp.ones((CW, D), jnp.float32)

  part1, degp = _conv1_deg(x, src2d, dst2d, z128, ones)

  h1, inv = pl.pallas_call(
      _tc1_body,
      grid=(GRID,),
      in_specs=[
          pl.BlockSpec((BLK, D), lambda i: (i, 0)),
          pl.BlockSpec((NC, BLK, D), lambda i: (0, i, 0)),
          pl.BlockSpec((NC, BLK, D), lambda i: (0, i, 0)),
          _full((D, D)),
          _full((D, D)),
          _full((1, D)),
          _full((1, D)),
          _full((1, D)),
      ],
      out_specs=[
          pl.BlockSpec((BLK, D), lambda i: (i, 0)),
          pl.BlockSpec((BLK, 1), lambda i: (i, 0)),
      ],
      out_shape=[
          jax.ShapeDtypeStruct((N, D), jnp.float32),
          jax.ShapeDtypeStruct((N, 1), jnp.float32),
      ],
  )(x, part1, degp, W_self0, W_nei0, b0.reshape(1, D), g0.reshape(1, D),
    be0.reshape(1, D))

  part2 = _conv(h1, src2d, dst2d, z128)

  out = pl.pallas_call(
      _tc2_body,
      grid=(GRID,),
      in_specs=[
          pl.BlockSpec((BLK, D), lambda i: (i, 0)),
          pl.BlockSpec((NC, BLK, D), lambda i: (0, i, 0)),
          pl.BlockSpec((BLK, 1), lambda i: (i, 0)),
          _full((D, D)),
          _full((D, D)),
          _full((1, D)),
          _full((1, D)),
          _full((1, D)),
          _full((D, D)),
          _full((1, D)),
          _full((D, D)),
          _full((1, D)),
          _full((D, 1)),
          _full((1, 1)),
      ],
      out_specs=pl.BlockSpec((BLK, 1), lambda i: (i, 0)),
      out_shape=jax.ShapeDtypeStruct((N, 1), jnp.float32),
  )(h1, part2, inv, W_self1, W_nei1, b1.reshape(1, D), g1.reshape(1, D),
    be1.reshape(1, D), W_lin0, bl0.reshape(1, D), W_lin1, bl1.reshape(1, D),
    W_head, b_head.reshape(1, 1))

  return out


# speedup vs baseline: 1.0272x; 1.0272x over previous
import functools

import jax
import jax.numpy as jnp
from jax import lax
from jax.experimental import pallas as pl
from jax.experimental.pallas import tpu as pltpu
from jax.experimental.pallas import tpu_sc as plsc

N = 10000
E = 320000
D = 128

NC = 2
NS = 16
NW = NC * NS
CW = 64
NCHUNK = 160
EPT = NCHUNK * CW
EP = EPT * NW
CWD = 120
NCHUNKD = 84
NA = N + 16
RZ = 624

_HIGH = jax.lax.Precision.DEFAULT

_MESH = plsc.VectorSubcoreMesh(core_axis_name="c", subcore_axis_name="s")


@functools.partial(
    pl.kernel,
    out_type=jax.ShapeDtypeStruct((NC, N, D), jnp.float32),
    mesh=_MESH,
    scratch_types=[
        [pltpu.VMEM((CW,), jnp.int32) for _ in range(8)],
        [pltpu.VMEM((CW,), jnp.int32) for _ in range(8)],
        pltpu.VMEM((4, CW, D), jnp.float32),
        pltpu.VMEM_SHARED((NA, D), jnp.float32),
        [pltpu.SemaphoreType.DMA for _ in range(8)],
        [pltpu.SemaphoreType.DMA for _ in range(4)],
        [pltpu.SemaphoreType.DMA for _ in range(4)],
    ],
)
def _conv(table, src2d, dst2d, zeros, out, sidx, didx, rows, agg_sh,
          isems, gsems, ssems):
  c = lax.axis_index("c")
  s = lax.axis_index("s")
  pltpu.sync_copy(zeros.at[pl.ds(s * RZ, RZ)], agg_sh.at[pl.ds(s * RZ, RZ)])

  @pl.when(s == 0)
  def _():
    pltpu.sync_copy(zeros.at[pl.ds(0, NA - NS * RZ)],
                    agg_sh.at[pl.ds(NS * RZ, NA - NS * RZ)])

  row0 = (c * NS + s) * NCHUNK

  def istart(j, b):
    pltpu.async_copy(src2d.at[row0 + j], sidx[b], isems[b])
    pltpu.async_copy(dst2d.at[row0 + j], didx[b], isems[b])

  def iwait(b):
    pltpu.make_async_copy(src2d.at[0], sidx[b], isems[b]).wait()
    pltpu.make_async_copy(dst2d.at[0], didx[b], isems[b]).wait()

  def gstart(b3, b6):
    pltpu.async_copy(table.at[sidx[b6]], rows.at[b3], gsems[b3])

  def gwait(b3):
    pltpu.make_async_copy(table.at[pl.ds(0, CW)], rows.at[b3],
                          gsems[b3]).wait()

  def sstart(b3, b6):
    pltpu.async_copy(rows.at[b3], agg_sh.at[didx[b6]], ssems[b3], add=True)

  def swait(b3):
    pltpu.make_async_copy(rows.at[b3], agg_sh.at[pl.ds(0, CW)],
                          ssems[b3]).wait()

  plsc.subcore_barrier()

  for j in range(5):
    istart(j, j)
  for j in range(3):
    iwait(j)
    gstart(j, j)

  def body(i, carry):
    j0 = 8 * i
    for b in range(8):
      j = j0 + b

      @pl.when(j + 3 < NCHUNK)
      def _():
        iwait((b + 3) % 8)
        @pl.when(j >= 1)
        def _():
          swait((b + 3) % 4)

        gstart((b + 3) % 4, (b + 3) % 8)

      gwait(b % 4)
      sstart(b % 4, b)

      @pl.when(j + 5 < NCHUNK)
      def _():
        istart(j + 5, (b + 5) % 8)
    return carry

  lax.fori_loop(0, NCHUNK // 8, body, 0)
  for b4 in range(4):
    swait(b4)

  plsc.subcore_barrier()
  pltpu.sync_copy(agg_sh.at[pl.ds(s * RZ, RZ)], out.at[c, pl.ds(s * RZ, RZ)])

  @pl.when(s == 0)
  def _():
    pltpu.sync_copy(agg_sh.at[pl.ds(NS * RZ, N - NS * RZ)],
                    out.at[c, pl.ds(NS * RZ, N - NS * RZ)])


@functools.partial(
    pl.kernel,
    out_type=jax.ShapeDtypeStruct((NC, N, D), jnp.float32),
    mesh=_MESH,
    scratch_types=[
        [pltpu.VMEM((CWD,), jnp.int32) for _ in range(4)],
        pltpu.VMEM((CWD, D), jnp.float32),
        pltpu.VMEM_SHARED((NA, D), jnp.float32),
        [pltpu.SemaphoreType.DMA for _ in range(4)],
        [pltpu.SemaphoreType.DMA for _ in range(4)],
    ],
)
def _degree(dst2d, ones, zeros, out, didx, ones_v, deg_sh, isems, ssems):
  c = lax.axis_index("c")
  s = lax.axis_index("s")
  pltpu.sync_copy(zeros.at[pl.ds(s * RZ, RZ)], deg_sh.at[pl.ds(s * RZ, RZ)])

  @pl.when(s == 0)
  def _():
    pltpu.sync_copy(zeros.at[pl.ds(0, NA - NS * RZ)],
                    deg_sh.at[pl.ds(NS * RZ, NA - NS * RZ)])

  row0 = (c * NS + s) * NCHUNKD
  pltpu.sync_copy(ones, ones_v)

  def istart(j, b):
    pltpu.async_copy(dst2d.at[row0 + j], didx[b], isems[b])

  def iwait(b):
    pltpu.make_async_copy(dst2d.at[0], didx[b], isems[b]).wait()

  def swait(b):
    pltpu.make_async_copy(ones_v, deg_sh.at[pl.ds(0, CWD)], ssems[b]).wait()

  plsc.subcore_barrier()
  istart(0, 0)

  def body(i, carry):
    j0 = 4 * i
    for b in range(4):
      j = j0 + b

      @pl.when(j + 1 < NCHUNKD)
      def _():
        @pl.when(j >= 3)
        def _():
          swait((b + 1) % 4)

        istart(j + 1, (b + 1) % 4)

      iwait(b)
      pltpu.async_copy(ones_v, deg_sh.at[didx[b]], ssems[b], add=True)
    return carry

  lax.fori_loop(0, NCHUNKD // 4, body, 0)
  for b in range(4):
    swait(b)

  plsc.subcore_barrier()
  pltpu.sync_copy(deg_sh.at[pl.ds(s * RZ, RZ)], out.at[c, pl.ds(s * RZ, RZ)])

  @pl.when(s == 0)
  def _():
    pltpu.sync_copy(deg_sh.at[pl.ds(NS * RZ, N - NS * RZ)],
                    out.at[c, pl.ds(NS * RZ, N - NS * RZ)])


BLK = 1000
GRID = N // BLK


def _tc1_body(h_ref, p_ref, deg_ref, Ws_ref, Wn_ref, b_ref, g_ref, be_ref,
              h1_ref, inv_ref):
  h = h_ref[...]
  p = p_ref[...]
  agg = p[0] + p[1]
  deg = deg_ref[0, :, :1] + deg_ref[1, :, :1]
  inv = 1.0 / jnp.maximum(deg, 1.0)
  z = (jnp.dot(h, Ws_ref[...], precision=_HIGH)
       + jnp.dot(agg * inv, Wn_ref[...], precision=_HIGH) + b_ref[...])
  mu = jnp.mean(z, axis=-1, keepdims=True)
  zc = z - mu
  var = jnp.mean(zc * zc, axis=-1, keepdims=True)
  zn = zc / jnp.sqrt(var + 1e-5) * g_ref[...] + be_ref[...]
  h1_ref[...] = jnp.maximum(zn, 0.0)
  inv_ref[...] = inv


def _tc2_body(h_ref, p_ref, inv_ref, Ws_ref, Wn_ref, b_ref, g_ref, be_ref,
              Wl0_ref, bl0_ref, Wl1_ref, bl1_ref, Wh_ref, bh_ref, out_ref):
  h = h_ref[...]
  p = p_ref[...]
  agg = (p[0] + p[1]) * inv_ref[...]
  z = (jnp.dot(h, Ws_ref[...], precision=_HIGH)
       + jnp.dot(agg, Wn_ref[...], precision=_HIGH) + b_ref[...])
  mu = jnp.mean(z, axis=-1, keepdims=True)
  zc = z - mu
  var = jnp.mean(zc * zc, axis=-1, keepdims=True)
  zn = zc / jnp.sqrt(var + 1e-5) * g_ref[...] + be_ref[...]
  h2 = jnp.maximum(zn, 0.0)
  z0 = jnp.maximum(jnp.dot(h2, Wl0_ref[...], precision=_HIGH)
                   + bl0_ref[...], 0.0)
  z1 = jnp.maximum(jnp.dot(z0, Wl1_ref[...], precision=_HIGH)
                   + bl1_ref[...], 0.0)
  out_ref[...] = jnp.dot(z1, Wh_ref[...], precision=_HIGH) + bh_ref[...]


def _full(shape):
  nd = len(shape)
  return pl.BlockSpec(shape, lambda i: (0,) * nd)


def kernel(x, edge_index, W_self0, W_nei0, b0, g0, be0, W_self1, W_nei1, b1,
           g1, be1, W_lin0, bl0, W_lin1, bl1, W_head, b_head):
  npad = EP - E
  pad_iota = jnp.arange(npad, dtype=jnp.int32)
  src_p = jnp.concatenate([edge_index[0], pad_iota % N])
  dst_p = jnp.concatenate([edge_index[1], N + (pad_iota % (NA - N))])
  src2d = src_p.reshape(EP // CW, CW)
  dst2d = dst_p.reshape(EP // CW, CW)
  dst2dd = dst_p.reshape(EP // CWD, CWD)
  z128 = jnp.zeros((N, D), jnp.float32)
  ones = jnp.ones((CWD, D), jnp.float32)

  degp = _degree(dst2dd, ones, z128)
  part1 = _conv(x, src2d, dst2d, z128)

  h1, inv = pl.pallas_call(
      _tc1_body,
      grid=(GRID,),
      in_specs=[
          pl.BlockSpec((BLK, D), lambda i: (i, 0)),
          pl.BlockSpec((NC, BLK, D), lambda i: (0, i, 0)),
          pl.BlockSpec((NC, BLK, D), lambda i: (0, i, 0)),
          _full((D, D)),
          _full((D, D)),
          _full((1, D)),
          _full((1, D)),
          _full((1, D)),
      ],
      out_specs=[
          pl.BlockSpec((BLK, D), lambda i: (i, 0)),
          pl.BlockSpec((BLK, 1), lambda i: (i, 0)),
      ],
      out_shape=[
          jax.ShapeDtypeStruct((N, D), jnp.float32),
          jax.ShapeDtypeStruct((N, 1), jnp.float32),
      ],
  )(x, part1, degp, W_self0, W_nei0, b0.reshape(1, D), g0.reshape(1, D),
    be0.reshape(1, D))

  part2 = _conv(h1, src2d, dst2d, z128)

  out = pl.pallas_call(
      _tc2_body,
      grid=(GRID,),
      in_specs=[
          pl.BlockSpec((BLK, D), lambda i: (i, 0)),
          pl.BlockSpec((NC, BLK, D), lambda i: (0, i, 0)),
          pl.BlockSpec((BLK, 1), lambda i: (i, 0)),
          _full((D, D)),
          _full((D, D)),
          _full((1, D)),
          _full((1, D)),
          _full((1, D)),
          _full((D, D)),
          _full((1, D)),
          _full((D, D)),
          _full((1, D)),
          _full((D, 1)),
          _full((1, 1)),
      ],
      out_specs=pl.BlockSpec((BLK, 1), lambda i: (i, 0)),
      out_shape=jax.ShapeDtypeStruct((N, 1), jnp.float32),
  )(h1, part2, inv, W_self1, W_nei1, b1.reshape(1, D), g1.reshape(1, D),
    be1.reshape(1, D), W_lin0, bl0.reshape(1, D), W_lin1, bl1.reshape(1, D),
    W_head, b_head.reshape(1, 1))

  return out


# speedup vs baseline: 1.0490x; 1.0212x over previous
import functools

import jax
import jax.numpy as jnp
from jax import lax
from jax.experimental import pallas as pl
from jax.experimental.pallas import tpu as pltpu
from jax.experimental.pallas import tpu_sc as plsc

N = 10000
E = 320000
D = 128

NC = 2
NS = 16
NW = NC * NS
CW = 64
NCHUNK = 160
EPT = NCHUNK * CW
EP = EPT * NW
NA = N + 16
RZ = 624

_PREC = jax.lax.Precision.DEFAULT

_MESH = plsc.VectorSubcoreMesh(core_axis_name="c", subcore_axis_name="s")


def _scratch():
  return [
      [pltpu.VMEM((CW,), jnp.int32) for _ in range(8)],
      [pltpu.VMEM((CW,), jnp.int32) for _ in range(8)],
      pltpu.VMEM((4, CW, D), jnp.float32),
      pltpu.VMEM_SHARED((NA, D), jnp.float32),
      [pltpu.SemaphoreType.DMA for _ in range(8)],
      [pltpu.SemaphoreType.DMA for _ in range(4)],
      [pltpu.SemaphoreType.DMA for _ in range(4)],
  ]


def _zero_acc(zeros, acc, s):
  pltpu.sync_copy(zeros.at[pl.ds(s * RZ, RZ)], acc.at[pl.ds(s * RZ, RZ)])

  @pl.when(s == 0)
  def _():
    pltpu.sync_copy(zeros.at[pl.ds(0, NA - NS * RZ)],
                    acc.at[pl.ds(NS * RZ, NA - NS * RZ)])


def _writeout(acc, out, c, s):
  pltpu.sync_copy(acc.at[pl.ds(s * RZ, RZ)], out.at[c, pl.ds(s * RZ, RZ)])

  @pl.when(s == 0)
  def _():
    pltpu.sync_copy(acc.at[pl.ds(NS * RZ, N - NS * RZ)],
                    out.at[c, pl.ds(NS * RZ, N - NS * RZ)])


def _conv_phase(table, src2d, dst2d, row0, sidx, didx, rows, agg_sh,
                isems, gsems, ssems):
  def istart(j, b):
    pltpu.async_copy(src2d.at[row0 + j], sidx[b], isems[b])
    pltpu.async_copy(dst2d.at[row0 + j], didx[b], isems[b])

  def iwait(b):
    pltpu.make_async_copy(src2d.at[0], sidx[b], isems[b]).wait()
    pltpu.make_async_copy(dst2d.at[0], didx[b], isems[b]).wait()

  def gstart(b4, b8):
    pltpu.async_copy(table.at[sidx[b8]], rows.at[b4], gsems[b4])

  def gwait(b4):
    pltpu.make_async_copy(table.at[pl.ds(0, CW)], rows.at[b4],
                          gsems[b4]).wait()

  def sstart(b4, b8):
    pltpu.async_copy(rows.at[b4], agg_sh.at[didx[b8]], ssems[b4], add=True)

  def swait(b4):
    pltpu.make_async_copy(rows.at[b4], agg_sh.at[pl.ds(0, CW)],
                          ssems[b4]).wait()

  for j in range(5):
    istart(j, j)
  for j in range(3):
    iwait(j)
    gstart(j, j)

  plsc.subcore_barrier()

  def body(i, carry):
    j0 = 8 * i
    for b in range(8):
      j = j0 + b

      @pl.when(j + 3 < NCHUNK)
      def _():
        iwait((b + 3) % 8)
        @pl.when(j >= 1)
        def _():
          swait((b + 3) % 4)

        gstart((b + 3) % 4, (b + 3) % 8)

      gwait(b % 4)
      sstart(b % 4, b)

      @pl.when(j + 5 < NCHUNK)
      def _():
        istart(j + 5, (b + 5) % 8)
    return carry

  lax.fori_loop(0, NCHUNK // 8, body, 0)
  for b4 in range(4):
    swait(b4)


@functools.partial(
    pl.kernel,
    out_type=[
        jax.ShapeDtypeStruct((NC, N, D), jnp.float32),
        jax.ShapeDtypeStruct((NC, N, D), jnp.float32),
    ],
    mesh=_MESH,
    scratch_types=_scratch(),
)
def _conv1_deg(table, src2d, dst2d, zeros, ones, out, out_deg, sidx, didx,
               rows, agg_sh, isems, gsems, ssems):
  c = lax.axis_index("c")
  s = lax.axis_index("s")
  row0 = (c * NS + s) * NCHUNK
  _zero_acc(zeros, agg_sh, s)
  pltpu.sync_copy(ones, rows.at[0])

  def istart_d(j, b):
    pltpu.async_copy(dst2d.at[row0 + j], didx[b], isems[b])

  def iwait_d(b):
    pltpu.make_async_copy(dst2d.at[0], didx[b], isems[b]).wait()

  def swait_d(b4):
    pltpu.make_async_copy(rows.at[0], agg_sh.at[pl.ds(0, CW)],
                          ssems[b4]).wait()

  for j in range(4):
    istart_d(j, j)
  plsc.subcore_barrier()

  def dbody(i, carry):
    j0 = 8 * i
    for b in range(8):
      j = j0 + b

      @pl.when(j >= 3)
      def _():
        swait_d((b + 1) % 4)

      iwait_d(b)
      pltpu.async_copy(rows.at[0], agg_sh.at[didx[b]], ssems[b % 4],
                       add=True)

      @pl.when(j + 4 < NCHUNK)
      def _():
        istart_d(j + 4, (b + 4) % 8)
    return carry

  lax.fori_loop(0, NCHUNK // 8, dbody, 0)
  for b4 in ((NCHUNK - 3) % 4, (NCHUNK - 2) % 4, (NCHUNK - 1) % 4):
    swait_d(b4)
  plsc.subcore_barrier()

  _writeout(agg_sh, out_deg, c, s)
  _zero_acc(zeros, agg_sh, s)

  _conv_phase(table, src2d, dst2d, row0, sidx, didx, rows, agg_sh,
              isems, gsems, ssems)

  plsc.subcore_barrier()
  _writeout(agg_sh, out, c, s)


@functools.partial(
    pl.kernel,
    out_type=jax.ShapeDtypeStruct((NC, N, D), jnp.float32),
    mesh=_MESH,
    scratch_types=_scratch(),
)
def _conv(table, src2d, dst2d, zeros, out, sidx, didx, rows, agg_sh,
          isems, gsems, ssems):
  c = lax.axis_index("c")
  s = lax.axis_index("s")
  row0 = (c * NS + s) * NCHUNK
  _zero_acc(zeros, agg_sh, s)
  _conv_phase(table, src2d, dst2d, row0, sidx, didx, rows, agg_sh,
              isems, gsems, ssems)
  plsc.subcore_barrier()
  _writeout(agg_sh, out, c, s)


BLK = 1000
GRID = N // BLK


def _tc1_body(h_ref, p_ref, deg_ref, Ws_ref, Wn_ref, b_ref, g_ref, be_ref,
              h1_ref, inv_ref):
  h = h_ref[...]
  p = p_ref[...]
  agg = p[0] + p[1]
  deg = deg_ref[0, :, :1] + deg_ref[1, :, :1]
  inv = 1.0 / jnp.maximum(deg, 1.0)
  z = (jnp.dot(h, Ws_ref[...], precision=_PREC)
       + jnp.dot(agg * inv, Wn_ref[...], precision=_PREC) + b_ref[...])
  mu = jnp.mean(z, axis=-1, keepdims=True)
  zc = z - mu
  var = jnp.mean(zc * zc, axis=-1, keepdims=True)
  zn = zc / jnp.sqrt(var + 1e-5) * g_ref[...] + be_ref[...]
  h1_ref[...] = jnp.maximum(zn, 0.0)
  inv_ref[...] = inv


def _tc2_body(h_ref, p_ref, inv_ref, Ws_ref, Wn_ref, b_ref, g_ref, be_ref,
              Wl0_ref, bl0_ref, Wl1_ref, bl1_ref, Wh_ref, bh_ref, out_ref):
  h = h_ref[...]
  p = p_ref[...]
  agg = (p[0] + p[1]) * inv_ref[...]
  z = (jnp.dot(h, Ws_ref[...], precision=_PREC)
       + jnp.dot(agg, Wn_ref[...], precision=_PREC) + b_ref[...])
  mu = jnp.mean(z, axis=-1, keepdims=True)
  zc = z - mu
  var = jnp.mean(zc * zc, axis=-1, keepdims=True)
  zn = zc / jnp.sqrt(var + 1e-5) * g_ref[...] + be_ref[...]
  h2 = jnp.maximum(zn, 0.0)
  z0 = jnp.maximum(jnp.dot(h2, Wl0_ref[...], precision=_PREC)
                   + bl0_ref[...], 0.0)
  z1 = jnp.maximum(jnp.dot(z0, Wl1_ref[...], precision=_PREC)
                   + bl1_ref[...], 0.0)
  out_ref[...] = jnp.dot(z1, Wh_ref[...], precision=_PREC) + bh_ref[...]


def _full(shape):
  nd = len(shape)
  return pl.BlockSpec(shape, lambda i: (0,) * nd)


def kernel(x, edge_index, W_self0, W_nei0, b0, g0, be0, W_self1, W_nei1, b1,
           g1, be1, W_lin0, bl0, W_lin1, bl1, W_head, b_head):
  npad = EP - E
  pad_iota = jnp.arange(npad, dtype=jnp.int32)
  src_p = jnp.concatenate([edge_index[0], pad_iota % N])
  dst_p = jnp.concatenate([edge_index[1], N + (pad_iota % (NA - N))])
  src2d = src_p.reshape(EP // CW, CW)
  dst2d = dst_p.reshape(EP // CW, CW)
  z128 = jnp.zeros((N, D), jnp.float32)
  ones = jnp.ones((CW, D), jnp.float32)

  part1, degp = _conv1_deg(x, src2d, dst2d, z128, ones)

  h1, inv = pl.pallas_call(
      _tc1_body,
      grid=(GRID,),
      in_specs=[
          pl.BlockSpec((BLK, D), lambda i: (i, 0)),
          pl.BlockSpec((NC, BLK, D), lambda i: (0, i, 0)),
          pl.BlockSpec((NC, BLK, D), lambda i: (0, i, 0)),
          _full((D, D)),
          _full((D, D)),
          _full((1, D)),
          _full((1, D)),
          _full((1, D)),
      ],
      out_specs=[
          pl.BlockSpec((BLK, D), lambda i: (i, 0)),
          pl.BlockSpec((BLK, 1), lambda i: (i, 0)),
      ],
      out_shape=[
          jax.ShapeDtypeStruct((N, D), jnp.float32),
          jax.ShapeDtypeStruct((N, 1), jnp.float32),
      ],
  )(x, part1, degp, W_self0, W_nei0, b0.reshape(1, D), g0.reshape(1, D),
    be0.reshape(1, D))

  part2 = _conv(h1, src2d, dst2d, z128)

  out = pl.pallas_call(
      _tc2_body,
      grid=(GRID,),
      in_specs=[
          pl.BlockSpec((BLK, D), lambda i: (i, 0)),
          pl.BlockSpec((NC, BLK, D), lambda i: (0, i, 0)),
          pl.BlockSpec((BLK, 1), lambda i: (i, 0)),
          _full((D, D)),
          _full((D, D)),
          _full((1, D)),
          _full((1, D)),
          _full((1, D)),
          _full((D, D)),
          _full((1, D)),
          _full((D, D)),
          _full((1, D)),
          _full((D, 1)),
          _full((1, 1)),
      ],
      out_specs=pl.BlockSpec((BLK, 1), lambda i: (i, 0)),
      out_shape=jax.ShapeDtypeStruct((N, 1), jnp.float32),
  )(h1, part2, inv, W_self1, W_nei1, b1.reshape(1, D), g1.reshape(1, D),
    be1.reshape(1, D), W_lin0, bl0.reshape(1, D), W_lin1, bl1.reshape(1, D),
    W_head, b_head.reshape(1, 1))

  return out
